# baseline (speedup 1.0000x reference)
import functools
import math

import jax
import jax.numpy as jnp
from jax import lax
from jax.experimental import pallas as pl
from jax.experimental.pallas import tpu as pltpu
from jax.experimental.pallas import tpu_sc as plsc

D_MODEL = 64
SCALE = math.sqrt(D_MODEL)
CHUNK = 640
GATHER_W = 128
NBUF = 2
VREGS_PER_ROW = D_MODEL // 16


@functools.lru_cache(maxsize=None)
def _make_kernel(B):
    info = plsc.get_sparse_core_info()
    NC, NS = info.num_cores, info.num_subcores
    NW = NC * NS
    b_per_w = B // NW
    nchunks = b_per_w // CHUNK
    ngroups = nchunks // NBUF
    nj = CHUNK // GATHER_W
    assert b_per_w * NW == B and nchunks * CHUNK == b_per_w
    assert ngroups * NBUF == nchunks and nj * GATHER_W == CHUNK

    mesh = plsc.VectorSubcoreMesh(core_axis_name="c", subcore_axis_name="s")

    @functools.partial(
        pl.kernel,
        out_type=jax.ShapeDtypeStruct((B, D_MODEL), jnp.float32),
        mesh=mesh,
        compiler_params=pltpu.CompilerParams(use_tc_tiling_on_sc=False),
        scratch_types=[
            pltpu.VMEM((NBUF, CHUNK), jnp.int32),
            pltpu.VMEM((NBUF, CHUNK, D_MODEL), jnp.float32),
            pltpu.SemaphoreType.DMA,
            pltpu.SemaphoreType.DMA,
            pltpu.SemaphoreType.DMA,
            pltpu.SemaphoreType.DMA,
        ],
    )
    def emb_kernel(x_hbm, lut_hbm, out_hbm, idx_v, rows_v, g0, g1, o0, o1):
        gsem = [g0, g1]
        osem = [o0, o1]
        wid = lax.axis_index("s") * NC + lax.axis_index("c")
        out0 = wid * b_per_w

        def fire(c, b):
            pltpu.sync_copy(
                x_hbm.at[pl.ds(out0 + c * CHUNK, CHUNK)], idx_v.at[b]
            )
            for j in range(nj):
                pltpu.async_copy(
                    lut_hbm.at[idx_v.at[b, pl.ds(j * GATHER_W, GATHER_W)]],
                    rows_v.at[b, pl.ds(j * GATHER_W, GATHER_W)],
                    gsem[b],
                )

        def wait_gathers(b):
            for j in range(nj):
                pltpu.make_async_copy(
                    lut_hbm.at[idx_v.at[b, pl.ds(j * GATHER_W, GATHER_W)]],
                    rows_v.at[b, pl.ds(j * GATHER_W, GATHER_W)],
                    gsem[b],
                ).wait()

        def scale(b):
            def body(i, carry):
                for u in range(4):
                    r = i * 4 + u
                    for k in range(VREGS_PER_ROW):
                        sl = pl.ds(k * 16, 16)
                        rows_v[b, r, sl] = rows_v[b, r, sl] * SCALE
                return carry
            lax.fori_loop(0, CHUNK // 4, body, 0)

        def fire_out(c, b):
            pltpu.async_copy(
                rows_v.at[b],
                out_hbm.at[pl.ds(out0 + c * CHUNK, CHUNK)],
                osem[b],
            )

        def wait_out(b):
            pltpu.make_async_copy(
                rows_v.at[b],
                out_hbm.at[pl.ds(out0, CHUNK)],
                osem[b],
            ).wait()

        for b in range(NBUF):
            fire(b, b)

        def group(g, carry):
            for b in range(NBUF):
                c = g * NBUF + b
                wait_gathers(b)
                scale(b)
                fire_out(c, b)
            for b in range(NBUF):
                c = g * NBUF + NBUF + b

                @pl.when(c < nchunks)
                def _():
                    wait_out(b)
                    fire(c, b)
            return carry

        lax.fori_loop(0, ngroups, group, 0)
        for b in range(NBUF):
            wait_out(b)

    return emb_kernel


def kernel(x, lut):
    orig_shape = x.shape
    B = x.size
    xf = x.astype(jnp.int32).reshape(B)
    out = _make_kernel(B)(xf, lut)
    return out.reshape(*orig_shape, D_MODEL)
